# Initial kernel scaffold; baseline (speedup 1.0000x reference)
import jax
import jax.numpy as jnp
from jax.experimental import pallas as pl


def kernel(embeds, adj_values, edge_index, W):
    raise NotImplementedError("write your pallas kernel here")


# speedup vs baseline: 4.4933x; 4.4933x over previous
import functools

import jax
import jax.numpy as jnp
from jax import lax
from jax.experimental import pallas as pl
from jax.experimental.pallas import tpu as pltpu
from jax.experimental.pallas import tpu_sc as plsc

N = 10000
E = 320000
D = 128

NC = 2
NS = 16
NW = NC * NS
EPW = E // NW
CHUNK = 80
NCHUNK = EPW // CHUNK
RCH = 80
NRCH = N // RCH
ZROWS = 16
LANES = 16


def _sc_aggregate(embeds, adj, src, dst):
    mesh = plsc.VectorSubcoreMesh(core_axis_name="c", subcore_axis_name="s")

    @functools.partial(
        pl.kernel,
        mesh=mesh,
        out_type=jax.ShapeDtypeStruct((NC, N, D), jnp.float32),
        scratch_types=[
            pltpu.VMEM((CHUNK,), jnp.int32),
            pltpu.VMEM((CHUNK,), jnp.int32),
            pltpu.VMEM((CHUNK,), jnp.float32),
            pltpu.VMEM((CHUNK, D), jnp.float32),
            pltpu.VMEM((ZROWS, D), jnp.float32),
            pltpu.VMEM_SHARED((N, D), jnp.float32),
            pltpu.SemaphoreType.DMA,
        ],
    )
    def body(embeds_hbm, adj_hbm, src_hbm, dst_hbm, out_hbm,
             sidx_v, didx_v, adj_v, rows_v, zbuf_v, acc_sh, sem):
        cid = lax.axis_index("c")
        sid = lax.axis_index("s")
        wid = cid * NS + sid

        zero16 = jnp.zeros((LANES,), jnp.float32)
        for i in range(ZROWS):
            for j in range(D // LANES):
                zbuf_v[i, pl.ds(LANES * j, LANES)] = zero16
        for k in range((NRCH + NS - 1) // NS):
            rc = sid + NS * k
            @pl.when(rc < NRCH)
            def _():
                for m in range(RCH // ZROWS):
                    pltpu.sync_copy(
                        zbuf_v, acc_sh.at[pl.ds(rc * RCH + m * ZROWS, ZROWS)])
        plsc.subcore_barrier()

        base_w = wid * EPW

        def chunk_body(ci, carry):
            base = base_w + ci * CHUNK
            pltpu.sync_copy(src_hbm.at[pl.ds(base, CHUNK)], sidx_v)
            pltpu.sync_copy(dst_hbm.at[pl.ds(base, CHUNK)], didx_v)
            pltpu.sync_copy(adj_hbm.at[pl.ds(base, CHUNK)], adj_v)
            pltpu.async_copy(embeds_hbm.at[sidx_v], rows_v, sem).wait()

            for g in range(CHUNK // LANES):
                a16 = adj_v[pl.ds(g * LANES, LANES)]
                for l in range(LANES):
                    e = g * LANES + l
                    av = jnp.full((LANES,), a16[l], jnp.float32)
                    for j in range(D // LANES):
                        sl = pl.ds(LANES * j, LANES)
                        rows_v[e, sl] = rows_v[e, sl] * av

            pltpu.sync_copy(rows_v, acc_sh.at[didx_v], add=True)
            return carry

        lax.fori_loop(0, NCHUNK, chunk_body, 0)

        plsc.subcore_barrier()
        for k in range((NRCH + NS - 1) // NS):
            rc = sid + NS * k
            @pl.when(rc < NRCH)
            def _():
                pltpu.sync_copy(acc_sh.at[pl.ds(rc * RCH, RCH)],
                                out_hbm.at[cid, pl.ds(rc * RCH, RCH)])

    return body(embeds, adj, src, dst)


def _tc_combine(p0, p1, W):
    BLK = 1000

    def body(p0_ref, p1_ref, w_ref, o_ref):
        x = p0_ref[...] + p1_ref[...]
        y = lax.dot_general(x, w_ref[...], (((1,), (1,)), ((), ())),
                            preferred_element_type=jnp.float32)
        o_ref[...] = jnp.where(y >= 0, y, 0.2 * y)

    return pl.pallas_call(
        body,
        grid=(N // BLK,),
        in_specs=[
            pl.BlockSpec((BLK, D), lambda i: (i, 0)),
            pl.BlockSpec((BLK, D), lambda i: (i, 0)),
            pl.BlockSpec((D, D), lambda i: (0, 0)),
        ],
        out_specs=pl.BlockSpec((BLK, D), lambda i: (i, 0)),
        out_shape=jax.ShapeDtypeStruct((N, D), jnp.float32),
    )(p0, p1, W)


def kernel(embeds, adj_values, edge_index, W):
    dst = edge_index[0].astype(jnp.int32)
    src = edge_index[1].astype(jnp.int32)
    partials = _sc_aggregate(embeds, adj_values, src, dst)
    return _tc_combine(partials[0], partials[1], W)


# speedup vs baseline: 10.9811x; 2.4439x over previous
import functools

import jax
import jax.numpy as jnp
from jax import lax
from jax.experimental import pallas as pl
from jax.experimental.pallas import tpu as pltpu
from jax.experimental.pallas import tpu_sc as plsc

N = 10000
E = 320000
D = 128

NC = 2
NS = 16
NW = NC * NS
EPW = E // NW
CHUNK = 40
NCHUNK = EPW // CHUNK
RCH = 40
NRCH = N // RCH
LANES = 16


def _sc_aggregate(embeds, src_flat, adj_flat, dst_flat):
    mesh = plsc.VectorSubcoreMesh(core_axis_name="c", subcore_axis_name="s")

    @functools.partial(
        pl.kernel,
        mesh=mesh,
        out_type=jax.ShapeDtypeStruct((NC, N, D), jnp.float32),
        scratch_types=[
            pltpu.VMEM((EPW,), jnp.int32),
            pltpu.VMEM((EPW,), jnp.float32),
            pltpu.VMEM((CHUNK,), jnp.int32),
            pltpu.VMEM((CHUNK,), jnp.int32),
            pltpu.VMEM((CHUNK, D), jnp.float32),
            pltpu.VMEM((CHUNK, D), jnp.float32),
            pltpu.VMEM_SHARED((N, D), jnp.float32),
            pltpu.SemaphoreType.DMA,
            pltpu.SemaphoreType.DMA,
            pltpu.SemaphoreType.DMA,
            pltpu.SemaphoreType.DMA,
        ],
    )
    def body(embeds_hbm, src_hbm, adj_hbm, dst_hbm, out_hbm,
             src_v, adj_v, db0_v, db1_v, rows0_v, rows1_v, acc_sh,
             semD0, semD1, semG0, semG1):
        cid = lax.axis_index("c")
        sid = lax.axis_index("s")
        wid = cid * NS + sid

        dbufs = (db0_v, db1_v)
        rbufs = (rows0_v, rows1_v)
        dsems = (semD0, semD1)
        gsems = (semG0, semG1)

        zero16 = jnp.zeros((LANES,), jnp.float32)
        for i in range(CHUNK):
            for j in range(D // LANES):
                rows0_v[i, pl.ds(LANES * j, LANES)] = zero16
        for k in range((NRCH + NS - 1) // NS):
            rc = sid + NS * k
            @pl.when(rc < NRCH)
            def _():
                pltpu.sync_copy(rows0_v, acc_sh.at[pl.ds(rc * RCH, RCH)])
        plsc.subcore_barrier()

        pltpu.sync_copy(src_hbm.at[pl.ds(wid * EPW, EPW)], src_v)
        pltpu.sync_copy(adj_hbm.at[pl.ds(wid * EPW, EPW)], adj_v)

        def dcopy(ci, b):
            d_off = wid * EPW + ci * CHUNK
            return pltpu.make_async_copy(
                dst_hbm.at[pl.ds(d_off, CHUNK)], dbufs[b], dsems[b])

        def gcopy(ci, b):
            idx = src_v.at[pl.ds(ci * CHUNK, CHUNK)]
            return pltpu.make_async_copy(
                embeds_hbm.at[idx], rbufs[b], gsems[b])

        def process(ci, b):
            buf = rbufs[b]
            for g in range((CHUNK + LANES - 1) // LANES):
                off = min(g * LANES, CHUNK - LANES)
                a16 = adj_v[pl.ds(ci * CHUNK + off, LANES)]
                lo = g * LANES
                hi = min(lo + LANES, CHUNK)
                for e in range(lo, hi):
                    av = jnp.full((LANES,), a16[e - off], jnp.float32)
                    for j in range(D // LANES):
                        sl = pl.ds(LANES * j, LANES)
                        buf[e, sl] = buf[e, sl] * av
            pltpu.sync_copy(buf, acc_sh.at[dbufs[b]], add=True)

        dcopy(0, 0).start()
        dcopy(1, 1).start()
        gcopy(0, 0).start()

        def pair_body(i, carry):
            c0 = 2 * i
            gcopy(c0 + 1, 1).start()
            gcopy(c0, 0).wait()
            dcopy(c0, 0).wait()
            process(c0, 0)

            @pl.when(c0 + 2 < NCHUNK)
            def _():
                dcopy(c0 + 2, 0).start()
                gcopy(c0 + 2, 0).start()

            gcopy(c0 + 1, 1).wait()
            dcopy(c0 + 1, 1).wait()
            process(c0 + 1, 1)

            @pl.when(c0 + 3 < NCHUNK)
            def _():
                dcopy(c0 + 3, 1).start()

            return carry

        lax.fori_loop(0, NCHUNK // 2, pair_body, 0)

        plsc.subcore_barrier()
        for k in range((NRCH + NS - 1) // NS):
            rc = sid + NS * k
            @pl.when(rc < NRCH)
            def _():
                pltpu.sync_copy(acc_sh.at[pl.ds(rc * RCH, RCH)],
                                out_hbm.at[cid, pl.ds(rc * RCH, RCH)])

    return body(embeds, src_flat, adj_flat, dst_flat)


def _tc_combine(p0, p1, W):
    BLK = 1000

    def body(p0_ref, p1_ref, w_ref, o_ref):
        x = p0_ref[...] + p1_ref[...]
        y = lax.dot_general(x, w_ref[...], (((1,), (1,)), ((), ())),
                            preferred_element_type=jnp.float32)
        o_ref[...] = jnp.where(y >= 0, y, 0.2 * y)

    return pl.pallas_call(
        body,
        grid=(N // BLK,),
        in_specs=[
            pl.BlockSpec((BLK, D), lambda i: (i, 0)),
            pl.BlockSpec((BLK, D), lambda i: (i, 0)),
            pl.BlockSpec((D, D), lambda i: (0, 0)),
        ],
        out_specs=pl.BlockSpec((BLK, D), lambda i: (i, 0)),
        out_shape=jax.ShapeDtypeStruct((N, D), jnp.float32),
    )(p0, p1, W)


def kernel(embeds, adj_values, edge_index, W):
    dst = edge_index[0].astype(jnp.int32)
    src = edge_index[1].astype(jnp.int32)
    partials = _sc_aggregate(embeds, src, adj_values, dst)
    return _tc_combine(partials[0], partials[1], W)
